# baseline (speedup 1.0000x reference)
import jax
import jax.numpy as jnp
from jax.experimental import pallas as pl

_K = 1024
_D = 256
_BLK = 256


def _vq_block(x_ref, wt_ref, w_ref, w2_ref, idx_ref, enc_ref, q_ref, loss_ref):
    i = pl.program_id(0)
    xb = x_ref[...]
    s = jnp.dot(xb, wt_ref[...], preferred_element_type=jnp.float32)
    x2 = jnp.sum(xb * xb, axis=1, keepdims=True)
    dist = (x2 + w2_ref[...]) - 2.0 * s
    m = jnp.min(dist, axis=1, keepdims=True)
    iota = jax.lax.broadcasted_iota(jnp.int32, (_BLK, _K), 1)
    idx = jnp.min(jnp.where(dist == m, iota, _K), axis=1)
    enc = (iota == idx[:, None]).astype(jnp.float32)
    enc_ref[...] = enc
    q = jnp.dot(enc, w_ref[...], preferred_element_type=jnp.float32)
    q_ref[...] = q
    idx_ref[...] = idx[:, None]

    part = jnp.sum((q - xb) ** 2).reshape(1, 1)

    @pl.when(i == 0)
    def _init():
        loss_ref[...] = jnp.zeros((1, 1), jnp.float32)

    loss_ref[...] += part


def kernel(x, W):
    B, C, H, Wd = x.shape
    flat_x = jnp.transpose(x, (0, 2, 3, 1)).reshape(-1, _D)
    n = flat_x.shape[0]
    wt = W.T
    w2 = jnp.sum(W ** 2, axis=1)[None, :]

    idx2, enc, q, loss_sum = pl.pallas_call(
        _vq_block,
        grid=(n // _BLK,),
        in_specs=[
            pl.BlockSpec((_BLK, _D), lambda i: (i, 0)),
            pl.BlockSpec((_D, _K), lambda i: (0, 0)),
            pl.BlockSpec((_K, _D), lambda i: (0, 0)),
            pl.BlockSpec((1, _K), lambda i: (0, 0)),
        ],
        out_specs=[
            pl.BlockSpec((_BLK, 1), lambda i: (i, 0)),
            pl.BlockSpec((_BLK, _K), lambda i: (i, 0)),
            pl.BlockSpec((_BLK, _D), lambda i: (i, 0)),
            pl.BlockSpec((1, 1), lambda i: (0, 0)),
        ],
        out_shape=[
            jax.ShapeDtypeStruct((n, 1), jnp.int32),
            jax.ShapeDtypeStruct((n, _K), jnp.float32),
            jax.ShapeDtypeStruct((n, _D), jnp.float32),
            jax.ShapeDtypeStruct((1, 1), jnp.float32),
        ],
    )(flat_x, wt, W, w2)

    loss = 2.0 * loss_sum[0, 0] / (n * _D)
    out = jnp.transpose(q.reshape(B, H, Wd, C), (0, 3, 1, 2))
    return (loss, out, enc, idx2.reshape(-1))
